# Initial kernel scaffold; baseline (speedup 1.0000x reference)
import jax
import jax.numpy as jnp
from jax.experimental import pallas as pl


def kernel(x0, x1, x2, x3, emb_word, emb_bigram, emb_trigram, W1, b1, W2, b2):
    raise NotImplementedError("write your pallas kernel here")


# speedup vs baseline: 1.3876x; 1.3876x over previous
import functools

import jax
import jax.numpy as jnp
from jax import lax
from jax.experimental import pallas as pl
from jax.experimental.pallas import tpu as pltpu
from jax.experimental.pallas import tpu_sc as plsc

B, L, E = 4096, 50, 200
H, C = 256, 20
NC, NS, LANES = 2, 16, 16
NW = NC * NS
BPW = B // NW
NK = E // LANES
TAIL = E - NK * LANES


def _sc_pool_body(x0_hbm, x2_hbm, x3_hbm, ew_hbm, eb_hbm, et_hbm, out_hbm,
                  idx0_v, idx2_v, idx3_v, rows0, rows1, rows2, pool_v,
                  sem0, sem1, sem2):
    wid = lax.axis_index("s") * NC + lax.axis_index("c")
    base = wid * BPW
    pltpu.sync_copy(x0_hbm.at[pl.ds(base, BPW)], idx0_v)
    pltpu.sync_copy(x2_hbm.at[pl.ds(base, BPW)], idx2_v)
    pltpu.sync_copy(x3_hbm.at[pl.ds(base, BPW)], idx3_v)

    tabs = ((ew_hbm, idx0_v, rows0, sem0),
            (eb_hbm, idx2_v, rows1, sem1),
            (et_hbm, idx3_v, rows2, sem2))

    iota = lax.iota(jnp.int32, LANES)
    tail_col = jnp.minimum(NK * LANES + iota, E - 1)
    tail_keep = iota < TAIL
    zero = jnp.zeros((LANES,), jnp.float32)

    for tab, idxv, rows, sem in tabs:
        pltpu.async_copy(tab.at[idxv.at[0]], rows, sem)

    def body(b, _):
        row_splat = jnp.full((LANES,), b, jnp.int32)
        for t, (tab, idxv, rows, sem) in enumerate(tabs):
            pltpu.make_async_copy(tab.at[idxv.at[0]], rows, sem).wait()

            def rbody(p, accs, rows=rows):
                new = list(accs)
                for rr in range(2):
                    r = p * 2 + rr
                    for k in range(NK):
                        new[k] = new[k] + rows[r, pl.ds(k * LANES, LANES)]
                    rs = jnp.full((LANES,), r, jnp.int32)
                    tail = plsc.load_gather(rows, [rs, tail_col])
                    new[NK] = new[NK] + jnp.where(tail_keep, tail, zero)
                return tuple(new)

            accs = lax.fori_loop(0, L // 2, rbody,
                                 tuple(zero for _ in range(NK + 1)))

            @pl.when(b + 1 < BPW)
            def _(tab=tab, idxv=idxv, rows=rows, sem=sem):
                pltpu.async_copy(tab.at[idxv.at[b + 1]], rows, sem)

            for k in range(NK):
                col = jnp.full((LANES,), t * E + k * LANES, jnp.int32) + iota
                plsc.store_scatter(pool_v, [row_splat, col], accs[k])
            colt = jnp.full((LANES,), t * E + NK * LANES, jnp.int32) + iota
            plsc.store_scatter(pool_v, [row_splat, colt], accs[NK],
                               mask=tail_keep)
        return ()

    lax.fori_loop(0, BPW, body, ())
    pltpu.sync_copy(pool_v, out_hbm.at[pl.ds(base, BPW)])


_sc_pool = pl.kernel(
    _sc_pool_body,
    out_type=jax.ShapeDtypeStruct((B, 3 * E), jnp.float32),
    mesh=plsc.VectorSubcoreMesh(core_axis_name="c", subcore_axis_name="s"),
    compiler_params=pltpu.CompilerParams(use_tc_tiling_on_sc=False,
                                         needs_layout_passes=False),
    scratch_types=[
        pltpu.VMEM((BPW, L), jnp.int32),
        pltpu.VMEM((BPW, L), jnp.int32),
        pltpu.VMEM((BPW, L), jnp.int32),
        pltpu.VMEM((L, E), jnp.float32),
        pltpu.VMEM((L, E), jnp.float32),
        pltpu.VMEM((L, E), jnp.float32),
        pltpu.VMEM((BPW, 3 * E), jnp.float32),
        pltpu.SemaphoreType.DMA,
        pltpu.SemaphoreType.DMA,
        pltpu.SemaphoreType.DMA,
    ],
)


def _mlp_body(x_ref, w1_ref, b1_ref, w2_ref, b2_ref, o_ref):
    x = x_ref[...] * (1.0 / L)
    h = jnp.dot(x, w1_ref[...], preferred_element_type=jnp.float32)
    h = jnp.maximum(h + b1_ref[...], 0.0)
    o = jnp.dot(h, w2_ref[...], preferred_element_type=jnp.float32)
    o_ref[...] = o + b2_ref[...]


def _tc_mlp(pooled, W1, b1, W2, b2):
    BM = 512
    return pl.pallas_call(
        _mlp_body,
        grid=(B // BM,),
        in_specs=[
            pl.BlockSpec((BM, 3 * E), lambda i: (i, 0)),
            pl.BlockSpec((3 * E, H), lambda i: (0, 0)),
            pl.BlockSpec((1, H), lambda i: (0, 0)),
            pl.BlockSpec((H, C), lambda i: (0, 0)),
            pl.BlockSpec((1, C), lambda i: (0, 0)),
        ],
        out_specs=pl.BlockSpec((BM, C), lambda i: (i, 0)),
        out_shape=jax.ShapeDtypeStruct((B, C), jnp.float32),
    )(pooled, W1, b1.reshape(1, H), W2, b2.reshape(1, C))


def kernel(x0, x1, x2, x3, emb_word, emb_bigram, emb_trigram, W1, b1, W2, b2):
    del x1
    pooled = _sc_pool(x0, x2, x3, emb_word, emb_bigram, emb_trigram)
    return _tc_mlp(pooled, W1, b1, W2, b2)


# speedup vs baseline: 1.3965x; 1.0064x over previous
import jax
import jax.numpy as jnp
from jax import lax
from jax.experimental import pallas as pl
from jax.experimental.pallas import tpu as pltpu
from jax.experimental.pallas import tpu_sc as plsc

B, L, E = 4096, 50, 200
H, C = 256, 20
NC, NS, LANES = 2, 16, 16
NW = NC * NS
BPW = B // NW
NK = E // LANES
TAIL = E - NK * LANES
G = 2
GI = G * L
NCH = BPW // G
NBUF = 4


def _sc_pool_body(x0r, x2r, x3r, ew_hbm, eb_hbm, et_hbm, o0, o1, o2,
                  idx_v, rows0, rows1, rows2, rows3, pool_v,
                  sem0, sem1, sem2, sem3):
    wid = lax.axis_index("s") * NC + lax.axis_index("c")
    base = wid * BPW

    iota = lax.iota(jnp.int32, LANES)
    tail_col = (NK * LANES + iota) % E
    tail_keep = iota < TAIL
    zero = jnp.zeros((LANES,), jnp.float32)
    bufs = ((rows0, sem0), (rows1, sem1), (rows2, sem2), (rows3, sem3))

    def chunk_compute(c, rows):
        for g in range(G):
            def rbody(p, accs, rows=rows, g=g):
                new = list(accs)
                for rr in range(2):
                    r = g * L + p * 2 + rr
                    for k in range(NK):
                        new[k] = new[k] + rows[r, pl.ds(k * LANES, LANES)]
                    rs = jnp.full((LANES,), r, jnp.int32)
                    tl = plsc.load_gather(rows, [rs, tail_col])
                    new[NK] = new[NK] + jnp.where(tail_keep, tl, zero)
                return tuple(new)

            accs = lax.fori_loop(0, L // 2, rbody,
                                 tuple(zero for _ in range(NK + 1)))
            rowi = jnp.full((LANES,), c * G + g, jnp.int32)
            for k in range(NK):
                plsc.store_scatter(pool_v, [rowi, k * LANES + iota], accs[k])
            plsc.store_scatter(pool_v, [rowi, NK * LANES + iota], accs[NK],
                               mask=tail_keep)

    for xr, tab, out in ((x0r, ew_hbm, o0), (x2r, eb_hbm, o1),
                         (x3r, et_hbm, o2)):
        pltpu.sync_copy(xr.at[pl.ds(wid * NCH, NCH)], idx_v)
        for q, (rows, sem) in enumerate(bufs):
            pltpu.async_copy(tab.at[idx_v.at[q]], rows, sem)

        def ring(j, _, tab=tab):
            for q, (rows, sem) in enumerate(bufs):
                c = j * NBUF + q
                pltpu.make_async_copy(tab.at[idx_v.at[0]], rows, sem).wait()
                chunk_compute(c, rows)

                @pl.when(c + NBUF < NCH)
                def _(rows=rows, sem=sem, c=c, tab=tab):
                    pltpu.async_copy(tab.at[idx_v.at[c + NBUF]], rows, sem)
            return ()

        lax.fori_loop(0, NCH // NBUF, ring, ())
        pltpu.sync_copy(pool_v, out.at[pl.ds(base, BPW)])


_sc_pool = pl.kernel(
    _sc_pool_body,
    out_type=(jax.ShapeDtypeStruct((B, E), jnp.float32),
              jax.ShapeDtypeStruct((B, E), jnp.float32),
              jax.ShapeDtypeStruct((B, E), jnp.float32)),
    mesh=plsc.VectorSubcoreMesh(core_axis_name="c", subcore_axis_name="s"),
    compiler_params=pltpu.CompilerParams(use_tc_tiling_on_sc=False,
                                         needs_layout_passes=False),
    scratch_types=[
        pltpu.VMEM((NCH, GI), jnp.int32),
        pltpu.VMEM((GI, E), jnp.float32),
        pltpu.VMEM((GI, E), jnp.float32),
        pltpu.VMEM((GI, E), jnp.float32),
        pltpu.VMEM((GI, E), jnp.float32),
        pltpu.VMEM((BPW, E), jnp.float32),
        pltpu.SemaphoreType.DMA,
        pltpu.SemaphoreType.DMA,
        pltpu.SemaphoreType.DMA,
        pltpu.SemaphoreType.DMA,
    ],
)


def _mlp_body(p0_ref, p1_ref, p2_ref, w1a_ref, w1b_ref, w1c_ref,
              b1_ref, w2_ref, b2_ref, o_ref):
    h = jnp.dot(p0_ref[...], w1a_ref[...], preferred_element_type=jnp.float32)
    h += jnp.dot(p1_ref[...], w1b_ref[...], preferred_element_type=jnp.float32)
    h += jnp.dot(p2_ref[...], w1c_ref[...], preferred_element_type=jnp.float32)
    h = jnp.maximum(h * (1.0 / L) + b1_ref[...], 0.0)
    o = jnp.dot(h, w2_ref[...], preferred_element_type=jnp.float32)
    o_ref[...] = o + b2_ref[...]


def _tc_mlp(p0, p1, p2, W1, b1, W2, b2):
    BM = 512
    full = lambda s: pl.BlockSpec(s, lambda i: (0, 0))
    return pl.pallas_call(
        _mlp_body,
        grid=(B // BM,),
        in_specs=[
            pl.BlockSpec((BM, E), lambda i: (i, 0)),
            pl.BlockSpec((BM, E), lambda i: (i, 0)),
            pl.BlockSpec((BM, E), lambda i: (i, 0)),
            full((E, H)), full((E, H)), full((E, H)),
            full((1, H)), full((H, C)), full((1, C)),
        ],
        out_specs=pl.BlockSpec((BM, C), lambda i: (i, 0)),
        out_shape=jax.ShapeDtypeStruct((B, C), jnp.float32),
    )(p0, p1, p2, W1[:E], W1[E:2 * E], W1[2 * E:],
      b1.reshape(1, H), W2, b2.reshape(1, C))


def kernel(x0, x1, x2, x3, emb_word, emb_bigram, emb_trigram, W1, b1, W2, b2):
    del x1
    p0, p1, p2 = _sc_pool(x0.reshape(NW * NCH, GI), x2.reshape(NW * NCH, GI),
                          x3.reshape(NW * NCH, GI),
                          emb_word, emb_bigram, emb_trigram)
    return _tc_mlp(p0, p1, p2, W1, b1, W2, b2)


# speedup vs baseline: 2.8086x; 2.0112x over previous
import jax
import jax.numpy as jnp
from jax import lax
from jax.experimental import pallas as pl
from jax.experimental.pallas import tpu as pltpu
from jax.experimental.pallas import tpu_sc as plsc

B, L, E = 4096, 50, 200
H, C = 256, 20
SEG = 128
TAILW = E - SEG
NC, NS, LANES = 2, 16, 16
NW = NC * NS
BPW = B // NW
NKA = SEG // LANES
NKB = 5
NBUF = 4


def _split_body(x_ref, a_ref, b_ref):
    x = x_ref[...]
    a_ref[...] = x[:, :SEG]
    b_ref[...] = jnp.concatenate(
        [x[:, SEG:], jnp.zeros((x.shape[0], SEG - TAILW), jnp.float32)],
        axis=1)


def _split_table(tab):
    V = tab.shape[0]
    R = 512
    grid = (pl.cdiv(V, R),)
    return pl.pallas_call(
        _split_body,
        grid=grid,
        in_specs=[pl.BlockSpec((R, E), lambda i: (i, 0))],
        out_specs=[pl.BlockSpec((R, SEG), lambda i: (i, 0)),
                   pl.BlockSpec((R, SEG), lambda i: (i, 0))],
        out_shape=[jax.ShapeDtypeStruct((V, SEG), jnp.float32),
                   jax.ShapeDtypeStruct((V, SEG), jnp.float32)],
    )(tab)


def _sc_pool_body(x_hbm, a_hbm, b_hbm, oa_hbm, ob_hbm,
                  idx_v, ra0, ra1, ra2, ra3, rb0, rb1, rb2, rb3,
                  pa_v, pb_v, sem0, sem1, sem2, sem3):
    wid = lax.axis_index("s") * NC + lax.axis_index("c")
    base = wid * BPW
    pltpu.sync_copy(x_hbm.at[pl.ds(base, BPW)], idx_v)

    iota = lax.iota(jnp.int32, LANES)
    zero = jnp.zeros((LANES,), jnp.float32)
    rings = ((ra0, rb0, sem0), (ra1, rb1, sem1),
             (ra2, rb2, sem2), (ra3, rb3, sem3))

    def zinit(b2, _):
        rowi = jnp.full((LANES,), b2, jnp.int32)
        for k in range(NKB, NKA):
            plsc.store_scatter(pb_v, [rowi, k * LANES + iota], zero)
        return ()
    lax.fori_loop(0, BPW, zinit, ())

    for q, (ra, rb, sem) in enumerate(rings):
        pltpu.async_copy(a_hbm.at[idx_v.at[q]], ra, sem)
        pltpu.async_copy(b_hbm.at[idx_v.at[q]], rb, sem)

    def ring(j, _):
        for q, (ra, rb, sem) in enumerate(rings):
            b = j * NBUF + q
            pltpu.make_async_copy(a_hbm.at[idx_v.at[0]], ra, sem).wait()
            pltpu.make_async_copy(b_hbm.at[idx_v.at[0]], rb, sem).wait()

            def rbody(p, accs, ra=ra, rb=rb):
                new = list(accs)
                for rr in range(2):
                    r = p * 2 + rr
                    for k in range(NKA):
                        new[k] = new[k] + ra[r, pl.ds(k * LANES, LANES)]
                    for k in range(NKB):
                        new[NKA + k] = (new[NKA + k]
                                        + rb[r, pl.ds(k * LANES, LANES)])
                return tuple(new)

            accs = lax.fori_loop(0, L // 2, rbody,
                                 tuple(zero for _ in range(NKA + NKB)))

            @pl.when(b + NBUF < BPW)
            def _(ra=ra, rb=rb, sem=sem, b=b):
                pltpu.async_copy(a_hbm.at[idx_v.at[b + NBUF]], ra, sem)
                pltpu.async_copy(b_hbm.at[idx_v.at[b + NBUF]], rb, sem)

            rowi = jnp.full((LANES,), b, jnp.int32)
            for k in range(NKA):
                plsc.store_scatter(pa_v, [rowi, k * LANES + iota], accs[k])
            for k in range(NKB):
                plsc.store_scatter(pb_v, [rowi, k * LANES + iota],
                                   accs[NKA + k])
        return ()

    lax.fori_loop(0, BPW // NBUF, ring, ())
    pltpu.sync_copy(pa_v, oa_hbm.at[pl.ds(base, BPW)])
    pltpu.sync_copy(pb_v, ob_hbm.at[pl.ds(base, BPW)])


def _make_sc_pool(V):
    return pl.kernel(
        _sc_pool_body,
        out_type=(jax.ShapeDtypeStruct((B, SEG), jnp.float32),
                  jax.ShapeDtypeStruct((B, SEG), jnp.float32)),
        mesh=plsc.VectorSubcoreMesh(core_axis_name="c", subcore_axis_name="s"),
        compiler_params=pltpu.CompilerParams(use_tc_tiling_on_sc=False,
                                             needs_layout_passes=False),
        scratch_types=(
            [pltpu.VMEM((BPW, L), jnp.int32)]
            + [pltpu.VMEM((L, SEG), jnp.float32) for _ in range(2 * NBUF)]
            + [pltpu.VMEM((BPW, SEG), jnp.float32) for _ in range(2)]
            + [pltpu.SemaphoreType.DMA for _ in range(NBUF)]),
    )


_sc_pool_word = _make_sc_pool(100000)
_sc_pool_ngram = _make_sc_pool(250499)


def _mlp_body(pa0, pb0, pa1, pb1, pa2, pb2,
              wa0, wb0, wa1, wb1, wa2, wb2, b1_ref, w2_ref, b2_ref, o_ref):
    f32 = jnp.float32
    h = jnp.dot(pa0[...], wa0[...], preferred_element_type=f32)
    h += jnp.dot(pb0[...], wb0[...], preferred_element_type=f32)
    h += jnp.dot(pa1[...], wa1[...], preferred_element_type=f32)
    h += jnp.dot(pb1[...], wb1[...], preferred_element_type=f32)
    h += jnp.dot(pa2[...], wa2[...], preferred_element_type=f32)
    h += jnp.dot(pb2[...], wb2[...], preferred_element_type=f32)
    h = jnp.maximum(h * (1.0 / L) + b1_ref[...], 0.0)
    o_ref[...] = jnp.dot(h, w2_ref[...], preferred_element_type=f32) + b2_ref[...]


def _tc_mlp(pools, W1, b1, W2, b2):
    BM = 512
    blk = pl.BlockSpec((BM, SEG), lambda i: (i, 0))
    full = lambda s: pl.BlockSpec(s, lambda i: (0, 0))
    ws = []
    for t in range(3):
        ws.append(W1[E * t:E * t + SEG])
        ws.append(jnp.pad(W1[E * t + SEG:E * (t + 1)],
                          ((0, SEG - TAILW), (0, 0))))
    return pl.pallas_call(
        _mlp_body,
        grid=(B // BM,),
        in_specs=([blk] * 6 + [full((SEG, H))] * 6
                  + [full((1, H)), full((H, C)), full((1, C))]),
        out_specs=pl.BlockSpec((BM, C), lambda i: (i, 0)),
        out_shape=jax.ShapeDtypeStruct((B, C), jnp.float32),
    )(*pools, *ws, b1.reshape(1, H), W2, b2.reshape(1, C))


def kernel(x0, x1, x2, x3, emb_word, emb_bigram, emb_trigram, W1, b1, W2, b2):
    del x1
    pools = []
    for x, tab, pool_fn in ((x0, emb_word, _sc_pool_word),
                            (x2, emb_bigram, _sc_pool_ngram),
                            (x3, emb_trigram, _sc_pool_ngram)):
        a, b = _split_table(tab)
        pools.extend(pool_fn(x, a, b))
    return _tc_mlp(pools, W1, b1, W2, b2)


# speedup vs baseline: 3.8887x; 1.3846x over previous
import jax
import jax.numpy as jnp
from jax import lax
from jax.experimental import pallas as pl
from jax.experimental.pallas import tpu as pltpu
from jax.experimental.pallas import tpu_sc as plsc

B, L, E = 4096, 50, 200
H, C = 256, 20
SEG = 128
TAILW = E - SEG
NC, NS, LANES = 2, 16, 16
NW = NC * NS
BPW = B // NW
NKA = SEG // LANES
NKB = 5
NBUF = 4


def _tail_body(x_ref, b_ref):
    x = x_ref[...]
    col = lax.broadcasted_iota(jnp.int32, x.shape, 1)
    b_ref[...] = jnp.where(col < TAILW, x, 0.0)


def _tail_table(tab):
    V = tab.shape[0]
    R = 1024
    return pl.pallas_call(
        _tail_body,
        grid=(pl.cdiv(V, R),),
        in_specs=[pl.BlockSpec((R, SEG), lambda i: (i, 1))],
        out_specs=pl.BlockSpec((R, SEG), lambda i: (i, 0)),
        out_shape=jax.ShapeDtypeStruct((V, SEG), jnp.float32),
    )(tab)


def _sc_pool_body(x_hbm, tab_hbm, b_hbm, oa_hbm, ob_hbm,
                  idx_v, ra0, ra1, ra2, ra3, rb0, rb1, rb2, rb3,
                  pa_v, pb_v, sem0, sem1, sem2, sem3):
    wid = lax.axis_index("s") * NC + lax.axis_index("c")
    base = wid * BPW
    pltpu.sync_copy(x_hbm.at[pl.ds(base, BPW)], idx_v)

    iota = lax.iota(jnp.int32, LANES)
    zero = jnp.zeros((LANES,), jnp.float32)
    rings = ((ra0, rb0, sem0), (ra1, rb1, sem1),
             (ra2, rb2, sem2), (ra3, rb3, sem3))

    def zinit(b2, _):
        rowi = jnp.full((LANES,), b2, jnp.int32)
        for k in range(NKB, NKA):
            plsc.store_scatter(pb_v, [rowi, k * LANES + iota], zero)
        return ()
    lax.fori_loop(0, BPW, zinit, ())

    def issue(b, ra, rb, sem):
        pltpu.async_copy(tab_hbm.at[idx_v.at[b], pl.ds(0, SEG)], ra, sem)
        pltpu.async_copy(b_hbm.at[idx_v.at[b]], rb, sem)

    for q, (ra, rb, sem) in enumerate(rings):
        issue(q, ra, rb, sem)

    def ring(j, _):
        for q, (ra, rb, sem) in enumerate(rings):
            b = j * NBUF + q
            pltpu.make_async_copy(tab_hbm.at[idx_v.at[0], pl.ds(0, SEG)],
                                  ra, sem).wait()
            pltpu.make_async_copy(b_hbm.at[idx_v.at[0]], rb, sem).wait()

            def rbody(p, accs, ra=ra, rb=rb):
                new = list(accs)
                for rr in range(2):
                    r = p * 2 + rr
                    for k in range(NKA):
                        new[k] = new[k] + ra[r, pl.ds(k * LANES, LANES)]
                    for k in range(NKB):
                        new[NKA + k] = (new[NKA + k]
                                        + rb[r, pl.ds(k * LANES, LANES)])
                return tuple(new)

            accs = lax.fori_loop(0, L // 2, rbody,
                                 tuple(zero for _ in range(NKA + NKB)))

            @pl.when(b + NBUF < BPW)
            def _(ra=ra, rb=rb, sem=sem, b=b):
                issue(b + NBUF, ra, rb, sem)

            rowi = jnp.full((LANES,), b, jnp.int32)
            for k in range(NKA):
                plsc.store_scatter(pa_v, [rowi, k * LANES + iota], accs[k])
            for k in range(NKB):
                plsc.store_scatter(pb_v, [rowi, k * LANES + iota],
                                   accs[NKA + k])
        return ()

    lax.fori_loop(0, BPW // NBUF, ring, ())
    pltpu.sync_copy(pa_v, oa_hbm.at[pl.ds(base, BPW)])
    pltpu.sync_copy(pb_v, ob_hbm.at[pl.ds(base, BPW)])


def _make_sc_pool(V):
    return pl.kernel(
        _sc_pool_body,
        out_type=(jax.ShapeDtypeStruct((B, SEG), jnp.float32),
                  jax.ShapeDtypeStruct((B, SEG), jnp.float32)),
        mesh=plsc.VectorSubcoreMesh(core_axis_name="c", subcore_axis_name="s"),
        compiler_params=pltpu.CompilerParams(use_tc_tiling_on_sc=True,
                                             needs_layout_passes=False),
        scratch_types=(
            [pltpu.VMEM((BPW, L), jnp.int32)]
            + [pltpu.VMEM((L, SEG), jnp.float32) for _ in range(2 * NBUF)]
            + [pltpu.VMEM((BPW, SEG), jnp.float32) for _ in range(2)]
            + [pltpu.SemaphoreType.DMA for _ in range(NBUF)]),
    )


_sc_pool_word = _make_sc_pool(100000)
_sc_pool_ngram = _make_sc_pool(250499)


def _mlp_body(pa0, pb0, pa1, pb1, pa2, pb2,
              wa0, wb0, wa1, wb1, wa2, wb2, b1_ref, w2_ref, b2_ref, o_ref):
    f32 = jnp.float32
    h = jnp.dot(pa0[...], wa0[...], preferred_element_type=f32)
    h += jnp.dot(pb0[...], wb0[...], preferred_element_type=f32)
    h += jnp.dot(pa1[...], wa1[...], preferred_element_type=f32)
    h += jnp.dot(pb1[...], wb1[...], preferred_element_type=f32)
    h += jnp.dot(pa2[...], wa2[...], preferred_element_type=f32)
    h += jnp.dot(pb2[...], wb2[...], preferred_element_type=f32)
    h = jnp.maximum(h * (1.0 / L) + b1_ref[...], 0.0)
    o_ref[...] = jnp.dot(h, w2_ref[...], preferred_element_type=f32) + b2_ref[...]


def _tc_mlp(pools, W1, b1, W2, b2):
    BM = 512
    blk = pl.BlockSpec((BM, SEG), lambda i: (i, 0))
    full = lambda s: pl.BlockSpec(s, lambda i: (0, 0))
    ws = []
    for t in range(3):
        ws.append(W1[E * t:E * t + SEG])
        ws.append(jnp.pad(W1[E * t + SEG:E * (t + 1)],
                          ((0, SEG - TAILW), (0, 0))))
    return pl.pallas_call(
        _mlp_body,
        grid=(B // BM,),
        in_specs=([blk] * 6 + [full((SEG, H))] * 6
                  + [full((1, H)), full((H, C)), full((1, C))]),
        out_specs=pl.BlockSpec((BM, C), lambda i: (i, 0)),
        out_shape=jax.ShapeDtypeStruct((B, C), jnp.float32),
    )(*pools, *ws, b1.reshape(1, H), W2, b2.reshape(1, C))


def kernel(x0, x1, x2, x3, emb_word, emb_bigram, emb_trigram, W1, b1, W2, b2):
    del x1
    pools = []
    for x, tab, pool_fn in ((x0, emb_word, _sc_pool_word),
                            (x2, emb_bigram, _sc_pool_ngram),
                            (x3, emb_trigram, _sc_pool_ngram)):
        tail = _tail_table(tab)
        pools.extend(pool_fn(x, tab, tail))
    return _tc_mlp(pools, W1, b1, W2, b2)


# speedup vs baseline: 4.3878x; 1.1283x over previous
import jax
import jax.numpy as jnp
from jax import lax
from jax.experimental import pallas as pl
from jax.experimental.pallas import tpu as pltpu
from jax.experimental.pallas import tpu_sc as plsc

B, L, E = 4096, 50, 200
H, C = 256, 20
SEG = 128
TAILW = E - SEG
NC, NS, LANES = 2, 16, 16
NW = NC * NS
BPW = B // NW
NKA = SEG // LANES
NKB = 5
NBUF = 4


def _tail_body(x_ref, b_ref):
    x = x_ref[...]
    col = lax.broadcasted_iota(jnp.int32, x.shape, 1)
    b_ref[...] = jnp.where(col < TAILW, x, 0.0)


def _tail_table(tab):
    V = tab.shape[0]
    R = 2048
    return pl.pallas_call(
        _tail_body,
        grid=(pl.cdiv(V, R),),
        in_specs=[pl.BlockSpec((R, SEG), lambda i: (i, 1))],
        out_specs=pl.BlockSpec((R, SEG), lambda i: (i, 0)),
        out_shape=jax.ShapeDtypeStruct((V, SEG), jnp.float32),
    )(tab)


def _sc_pool_body(x_hbm, tab_hbm, b_hbm, oa_hbm, ob_hbm,
                  idx_v, ra0, ra1, ra2, ra3, rb0, rb1, rb2, rb3,
                  pa_v, pb_v, sem0, sem1, sem2, sem3):
    wid = lax.axis_index("s") * NC + lax.axis_index("c")
    base = wid * BPW
    pltpu.sync_copy(x_hbm.at[pl.ds(base, BPW)], idx_v)

    iota = lax.iota(jnp.int32, LANES)
    zero = jnp.zeros((LANES,), jnp.float32)
    rings = ((ra0, rb0, sem0), (ra1, rb1, sem1),
             (ra2, rb2, sem2), (ra3, rb3, sem3))

    def zinit(b2, _):
        rowi = jnp.full((LANES,), b2, jnp.int32)
        for k in range(NKB, NKA):
            plsc.store_scatter(pb_v, [rowi, k * LANES + iota], zero)
        return ()
    lax.fori_loop(0, BPW, zinit, ())

    def issue(b, ra, rb, sem):
        pltpu.async_copy(tab_hbm.at[idx_v.at[b], pl.ds(0, SEG)], ra, sem)
        pltpu.async_copy(b_hbm.at[idx_v.at[b]], rb, sem)

    for q, (ra, rb, sem) in enumerate(rings):
        issue(q, ra, rb, sem)

    def ring(j, _):
        for q, (ra, rb, sem) in enumerate(rings):
            b = j * NBUF + q
            pltpu.make_async_copy(tab_hbm.at[idx_v.at[0], pl.ds(0, SEG)],
                                  ra, sem).wait()
            pltpu.make_async_copy(b_hbm.at[idx_v.at[0]], rb, sem).wait()

            def rbody(p, accs, ra=ra, rb=rb):
                new = list(accs)
                for rr in range(2):
                    r = p * 2 + rr
                    for k in range(NKA):
                        new[k] = new[k] + ra[r, pl.ds(k * LANES, LANES)]
                    for k in range(NKB):
                        new[NKA + k] = (new[NKA + k]
                                        + rb[r, pl.ds(k * LANES, LANES)])
                return tuple(new)

            accs = lax.fori_loop(0, L // 2, rbody,
                                 tuple(zero for _ in range(NKA + NKB)))

            @pl.when(b + NBUF < BPW)
            def _(ra=ra, rb=rb, sem=sem, b=b):
                issue(b + NBUF, ra, rb, sem)

            rowi = jnp.full((LANES,), b, jnp.int32)
            for k in range(NKA):
                plsc.store_scatter(pa_v, [rowi, k * LANES + iota], accs[k])
            for k in range(NKB):
                plsc.store_scatter(pb_v, [rowi, k * LANES + iota],
                                   accs[NKA + k])
        return ()

    lax.fori_loop(0, BPW // NBUF, ring, ())
    pltpu.sync_copy(pa_v, oa_hbm.at[pl.ds(base, BPW)])
    pltpu.sync_copy(pb_v, ob_hbm.at[pl.ds(base, BPW)])


def _make_sc_pool(V):
    return pl.kernel(
        _sc_pool_body,
        out_type=(jax.ShapeDtypeStruct((B, SEG), jnp.float32),
                  jax.ShapeDtypeStruct((B, SEG), jnp.float32)),
        mesh=plsc.VectorSubcoreMesh(core_axis_name="c", subcore_axis_name="s"),
        compiler_params=pltpu.CompilerParams(use_tc_tiling_on_sc=True,
                                             needs_layout_passes=False),
        scratch_types=(
            [pltpu.VMEM((BPW, L), jnp.int32)]
            + [pltpu.VMEM((L, SEG), jnp.float32) for _ in range(2 * NBUF)]
            + [pltpu.VMEM((BPW, SEG), jnp.float32) for _ in range(2)]
            + [pltpu.SemaphoreType.DMA for _ in range(NBUF)]),
    )


_sc_pool_word = _make_sc_pool(100000)
_sc_pool_ngram = _make_sc_pool(250499)


def _mlp_body(pa0, pb0, pa1, pb1, pa2, pb2,
              wa0, wb0, wa1, wb1, wa2, wb2, b1_ref, w2_ref, b2_ref, o_ref):
    f32 = jnp.float32
    h = jnp.dot(pa0[...], wa0[...], preferred_element_type=f32)
    h += jnp.dot(pb0[...], wb0[...], preferred_element_type=f32)
    h += jnp.dot(pa1[...], wa1[...], preferred_element_type=f32)
    h += jnp.dot(pb1[...], wb1[...], preferred_element_type=f32)
    h += jnp.dot(pa2[...], wa2[...], preferred_element_type=f32)
    h += jnp.dot(pb2[...], wb2[...], preferred_element_type=f32)
    h = jnp.maximum(h * (1.0 / L) + b1_ref[...], 0.0)
    o_ref[...] = jnp.dot(h, w2_ref[...], preferred_element_type=f32) + b2_ref[...]


def _tc_mlp(pools, W1, b1, W2, b2):
    BM = 512
    blk = pl.BlockSpec((BM, SEG), lambda i: (i, 0))
    full = lambda s: pl.BlockSpec(s, lambda i: (0, 0))
    ws = []
    for t in range(3):
        ws.append(W1[E * t:E * t + SEG])
        ws.append(jnp.pad(W1[E * t + SEG:E * (t + 1)],
                          ((0, SEG - TAILW), (0, 0))))
    return pl.pallas_call(
        _mlp_body,
        grid=(B // BM,),
        in_specs=([blk] * 6 + [full((SEG, H))] * 6
                  + [full((1, H)), full((H, C)), full((1, C))]),
        out_specs=pl.BlockSpec((BM, C), lambda i: (i, 0)),
        out_shape=jax.ShapeDtypeStruct((B, C), jnp.float32),
    )(*pools, *ws, b1.reshape(1, H), W2, b2.reshape(1, C))


def kernel(x0, x1, x2, x3, emb_word, emb_bigram, emb_trigram, W1, b1, W2, b2):
    del x1
    work = ((x0, emb_word, _sc_pool_word),
            (x2, emb_bigram, _sc_pool_ngram),
            (x3, emb_trigram, _sc_pool_ngram))
    tails = [_tail_table(tab) for _, tab, _ in work]
    pools = []
    for (x, tab, pool_fn), tail in zip(work, tails):
        pools.extend(pool_fn(x, tab, tail))
    return _tc_mlp(pools, W1, b1, W2, b2)
